# baseline (speedup 1.0000x reference)
import jax
import jax.numpy as jnp
from jax import lax
from jax.experimental import pallas as pl
from jax.experimental.pallas import tpu as pltpu
from jax.experimental.pallas import tpu_sc as plsc

N_NODES = 10000
D = 128
DH = D // 2
N_EDGES = 320000
MIN_NORM = 1e-15
BALL_EPS = 4e-3

NC = 2
NS = 16
L = 16

N_PAD = 10240
N_ACC = 10112
ROWS_PER_TILE = N_ACC // NS

B = 128
RPT_TAIL = ROWS_PER_TILE % B
EPS_RAW = N_EDGES // NS
G = 32
NCHUNK = 5
NBLK = G * NCHUNK
EPS = NBLK * B


def _prep_body(x_ref, w_ref, z_ref, st_ref):
    xb = x_ref[...]
    nsq = jnp.sum(xb * xb, axis=1, keepdims=True)
    norm = jnp.maximum(jnp.sqrt(nsq), MIN_NORM)
    a = jnp.clip(norm, -1.0 + 1e-7, 1.0 - 1e-7)
    artanh = 0.5 * (jnp.log1p(a) - jnp.log1p(-a))
    z = (artanh / norm) * xb
    zpad = jnp.zeros((N_PAD - N_NODES, D), jnp.float32)
    z_ref[...] = jnp.concatenate([z, zpad], axis=0)
    w = w_ref[...]
    s = jnp.sum(z * w[:, :D], axis=1)
    t = jnp.sum(z * w[:, D:], axis=1)
    spad = jnp.zeros((2, N_PAD - N_NODES), jnp.float32)
    st_ref[...] = jnp.concatenate(
        [jnp.stack([s, t], axis=0), spad], axis=1)


def _tc_prep(x, attn_w):
    return pl.pallas_call(
        _prep_body,
        out_shape=(
            jax.ShapeDtypeStruct((N_PAD, D), jnp.float32),
            jax.ShapeDtypeStruct((2, N_PAD), jnp.float32),
        ),
    )(x, attn_w)


def _sc_body(src_hbm, dst_hbm, st_hbm, z_hbm, acc_out, den_out,
             stab, ttab, srcv, dstv, gidx, ev, zrows, stage, zbufd, acc, accd,
             gsem, ssem0, ssem1, dsem):
    c = lax.axis_index("c")
    s = lax.axis_index("s")

    pltpu.sync_copy(st_hbm.at[0], stab)
    pltpu.sync_copy(st_hbm.at[1], ttab)

    coff = lax.broadcast(c, (L,))

    zeros16 = jnp.zeros((L,), jnp.float32)

    def zrow_zero(i, carry):
        for k in range(DH // L):
            stage[0, i, pl.ds(k * L, L)] = zeros16
        return carry

    lax.fori_loop(0, B, zrow_zero, 0)

    def zd_zero(i, carry):
        zbufd[pl.ds(i * L, L)] = zeros16
        return carry

    lax.fori_loop(0, (ROWS_PER_TILE + L - 1) // L, zd_zero, 0)

    for k in range(ROWS_PER_TILE // B):
        pltpu.sync_copy(stage.at[0],
                        acc.at[pl.ds(s * ROWS_PER_TILE + k * B, B)])
    pltpu.sync_copy(
        stage.at[0].at[pl.ds(0, RPT_TAIL)],
        acc.at[pl.ds(s * ROWS_PER_TILE + (ROWS_PER_TILE // B) * B, RPT_TAIL)])
    pltpu.sync_copy(zbufd.at[pl.ds(0, ROWS_PER_TILE)],
                    accd.at[pl.ds(s * ROWS_PER_TILE, ROWS_PER_TILE)])
    plsc.subcore_barrier()

    def chunk_body(g, carry):
        pltpu.sync_copy(src_hbm.at[s].at[pl.ds(g * G, G)], srcv)
        pltpu.sync_copy(dst_hbm.at[s].at[pl.ds(g * G, G)], dstv)

        def gid_body(i, carry2):
            for k in range(B // L):
                sl = pl.ds(k * L, L)
                sv2 = srcv[i, sl]
                gidx[i, sl] = sv2 + sv2 + coff
            return carry2

        lax.fori_loop(0, G, gid_body, 0)

        pltpu.async_copy(z_hbm.at[gidx.at[0]], zrows.at[0], gsem)
        pltpu.async_copy(z_hbm.at[gidx.at[1]], zrows.at[1], gsem)
        pltpu.async_copy(z_hbm.at[gidx.at[2]], zrows.at[2], gsem)

        def blk_body(b, carry2):
            cur = lax.rem(b, 5)
            pltpu.make_async_copy(
                z_hbm.at[gidx.at[b]], zrows.at[cur], gsem).wait()

            @pl.when(b >= 2)
            def _():
                drain = acc.at[dstv.at[b]]

                @pl.when(lax.rem(b, 2) == 0)
                def _():
                    pltpu.make_async_copy(
                        stage.at[0], drain, ssem0).wait()

                @pl.when(lax.rem(b, 2) == 1)
                def _():
                    pltpu.make_async_copy(
                        stage.at[1], drain, ssem1).wait()

            @pl.when(b + 3 < G)
            def _():
                pltpu.async_copy(
                    z_hbm.at[gidx.at[b + 3]], zrows.at[lax.rem(b + 3, 5)],
                    gsem)

            sb = lax.rem(b, 2)

            for k in range(B // L):
                sl = pl.ds(k * L, L)
                si = srcv[b, sl]
                di = dstv[b, sl]
                sv = plsc.load_gather(stab, [si])
                tv = plsc.load_gather(ttab, [di])
                e16 = jnp.exp(sv + tv)
                ev[b, sl] = e16
                for j in range(L):
                    av = e16.at[jnp.full((L,), j, jnp.int32)].get(
                        mode="promise_in_bounds")
                    r = k * L + j
                    for m in range(DH // (2 * L)):
                        xx = zrows[cur, r, pl.ds(m * 2 * L, 2 * L)]
                        lo, hi = plsc.unpack(
                            xx, format=plsc.PackFormat.INTERLEAVED,
                            preferred_element_type=jnp.float32)
                        stage[sb, r, pl.ds(m * 2 * L, L)] = lo * av
                        stage[sb, r, pl.ds(m * 2 * L + L, L)] = hi * av

            @pl.when(sb == 0)
            def _():
                pltpu.async_copy(
                    stage.at[0], acc.at[dstv.at[b]], ssem0, add=True)

            @pl.when(sb == 1)
            def _():
                pltpu.async_copy(
                    stage.at[1], acc.at[dstv.at[b]], ssem1, add=True)

            @pl.when(c == 0)
            def _():
                pltpu.async_copy(ev.at[b], accd.at[dstv.at[b]], dsem,
                                 add=True)

            return carry2

        lax.fori_loop(0, G, blk_body, 0)

        @pl.when(c == 0)
        def _():
            pltpu.make_async_copy(src_hbm.at[s].at[pl.ds(0, G)],
                                  gidx, dsem).wait()

        pltpu.make_async_copy(
            stage.at[0], acc.at[dstv.at[0]], ssem0).wait()
        pltpu.make_async_copy(
            stage.at[1], acc.at[dstv.at[0]], ssem1).wait()
        return carry

    lax.fori_loop(0, NCHUNK, chunk_body, 0)
    plsc.subcore_barrier()

    rs = pl.ds(s * ROWS_PER_TILE, ROWS_PER_TILE)
    pltpu.sync_copy(acc.at[rs], acc_out.at[c].at[rs])

    @pl.when(c == 0)
    def _():
        pltpu.sync_copy(accd.at[rs], den_out.at[rs])


def _sc_agg(src3, dst3, st, zflat):
    mesh = plsc.VectorSubcoreMesh(
        core_axis_name="c", subcore_axis_name="s", num_cores=NC,
        num_subcores=NS)
    f = pl.kernel(
        _sc_body,
        out_type=(
            jax.ShapeDtypeStruct((NC, N_ACC, DH), jnp.float32),
            jax.ShapeDtypeStruct((N_ACC,), jnp.float32),
        ),
        mesh=mesh,
        compiler_params=pltpu.CompilerParams(
            needs_layout_passes=False, use_tc_tiling_on_sc=False),
        scratch_types=[
            pltpu.VMEM((N_PAD,), jnp.float32),
            pltpu.VMEM((N_PAD,), jnp.float32),
            pltpu.VMEM((G, B), jnp.int32),
            pltpu.VMEM((G, B), jnp.int32),
            pltpu.VMEM((G, B), jnp.int32),
            pltpu.VMEM((G, B), jnp.float32),
            pltpu.VMEM((5, B, DH), jnp.bfloat16),
            pltpu.VMEM((2, B, DH), jnp.float32),
            pltpu.VMEM((((ROWS_PER_TILE + L - 1) // L) * L,), jnp.float32),
            pltpu.VMEM_SHARED((N_ACC, DH), jnp.float32),
            pltpu.VMEM_SHARED((N_ACC,), jnp.float32),
            pltpu.SemaphoreType.DMA,
            pltpu.SemaphoreType.DMA,
            pltpu.SemaphoreType.DMA,
            pltpu.SemaphoreType.DMA,
        ],
    )
    return f(src3, dst3, st, zflat)


def _fin_body(acc_ref, den_ref, out_ref):
    a = jnp.concatenate(
        [acc_ref[0, :N_NODES], acc_ref[1, :N_NODES]], axis=1)
    d = den_ref[:N_NODES]
    h = a / jnp.maximum(d, 1e-16)[:, None]
    un = jnp.maximum(
        jnp.sqrt(jnp.sum(h * h, axis=1, keepdims=True)), MIN_NORM)
    o1 = jnp.tanh(un) * h / un
    n1 = jnp.maximum(
        jnp.sqrt(jnp.sum(o1 * o1, axis=1, keepdims=True)), MIN_NORM)
    maxnorm = 1.0 - BALL_EPS
    out_ref[...] = jnp.where(n1 > maxnorm, o1 / n1 * maxnorm, o1)


def _tc_finish(acc, den):
    return pl.pallas_call(
        _fin_body,
        out_shape=jax.ShapeDtypeStruct((N_NODES, D), jnp.float32),
    )(acc, den)


@jax.jit
def kernel(x, edge_index, attn_w):
    z, st = _tc_prep(x, attn_w)
    zb = (z.astype(jnp.bfloat16)
          .reshape(N_PAD, D // (2 * L), 2, L)
          .transpose(0, 1, 3, 2)
          .reshape(N_PAD, D))
    zflat = zb.reshape(NC * N_PAD, DH)

    pad = N_NODES + (jnp.arange(EPS - EPS_RAW, dtype=jnp.int32)
                     % (N_ACC - N_NODES))
    pad2 = jnp.broadcast_to(pad, (NS, EPS - EPS_RAW))
    src3 = jnp.concatenate(
        [edge_index[0].reshape(NS, EPS_RAW), pad2], axis=1
    ).reshape(NS, NBLK, B)
    dst3 = jnp.concatenate(
        [edge_index[1].reshape(NS, EPS_RAW), pad2], axis=1
    ).reshape(NS, NBLK, B)

    acc, den = _sc_agg(src3, dst3, st, zflat)
    return _tc_finish(acc, den)


# speedup vs baseline: 4.4885x; 4.4885x over previous
import jax
import jax.numpy as jnp
from jax import lax
from jax.experimental import pallas as pl
from jax.experimental.pallas import tpu as pltpu
from jax.experimental.pallas import tpu_sc as plsc

N_NODES = 10000
D = 128
DH = D // 2
N_EDGES = 320000
MIN_NORM = 1e-15
BALL_EPS = 4e-3

NC = 2
NS = 16
L = 16

N_PAD = 10240
N_ACC = 10112
ROWS_PER_TILE = N_ACC // NS

B = 128
RPT_TAIL = ROWS_PER_TILE % B
EPS_RAW = N_EDGES // NS
G = 32
NCHUNK = 5
NBLK = G * NCHUNK
EPS = NBLK * B


def _prep_body(x_ref, w_ref, z_ref, st_ref):
    xb = x_ref[...]
    nsq = jnp.sum(xb * xb, axis=1, keepdims=True)
    norm = jnp.maximum(jnp.sqrt(nsq), MIN_NORM)
    a = jnp.clip(norm, -1.0 + 1e-7, 1.0 - 1e-7)
    artanh = 0.5 * (jnp.log1p(a) - jnp.log1p(-a))
    z = (artanh / norm) * xb
    zpad = jnp.zeros((N_PAD - N_NODES, D), jnp.float32)
    z_ref[...] = jnp.concatenate([z, zpad], axis=0)
    w = w_ref[...]
    s = jnp.sum(z * w[:, :D], axis=1)
    t = jnp.sum(z * w[:, D:], axis=1)
    spad = jnp.zeros((2, N_PAD - N_NODES), jnp.float32)
    st_ref[...] = jnp.concatenate(
        [jnp.stack([s, t], axis=0), spad], axis=1)


def _tc_prep(x, attn_w):
    return pl.pallas_call(
        _prep_body,
        out_shape=(
            jax.ShapeDtypeStruct((N_PAD, D), jnp.float32),
            jax.ShapeDtypeStruct((2, N_PAD), jnp.float32),
        ),
    )(x, attn_w)


def _sc_body(src_hbm, dst_hbm, st_hbm, z_hbm, acc_out, den_out,
             stab, ttab, srcv, dstv, gidx, ev, zrows, zbufd, acc, accd,
             gsem, ssem0, ssem1, dsem):
    c = lax.axis_index("c")
    s = lax.axis_index("s")

    pltpu.sync_copy(st_hbm.at[0], stab)
    pltpu.sync_copy(st_hbm.at[1], ttab)

    coff = lax.broadcast(c, (L,))

    zeros16 = jnp.zeros((L,), jnp.float32)

    def zrow_zero(i, carry):
        for k in range(DH // L):
            zrows[0, i, pl.ds(k * L, L)] = zeros16
        return carry

    lax.fori_loop(0, B, zrow_zero, 0)

    def zd_zero(i, carry):
        zbufd[pl.ds(i * L, L)] = zeros16
        return carry

    lax.fori_loop(0, (ROWS_PER_TILE + L - 1) // L, zd_zero, 0)

    for k in range(ROWS_PER_TILE // B):
        pltpu.sync_copy(zrows.at[0],
                        acc.at[pl.ds(s * ROWS_PER_TILE + k * B, B)])
    pltpu.sync_copy(
        zrows.at[0].at[pl.ds(0, RPT_TAIL)],
        acc.at[pl.ds(s * ROWS_PER_TILE + (ROWS_PER_TILE // B) * B, RPT_TAIL)])
    pltpu.sync_copy(zbufd.at[pl.ds(0, ROWS_PER_TILE)],
                    accd.at[pl.ds(s * ROWS_PER_TILE, ROWS_PER_TILE)])
    plsc.subcore_barrier()

    def chunk_body(g, carry):
        pltpu.sync_copy(src_hbm.at[s].at[pl.ds(g * G, G)], srcv)
        pltpu.sync_copy(dst_hbm.at[s].at[pl.ds(g * G, G)], dstv)

        def gid_body(i, carry2):
            for k in range(B // L):
                sl = pl.ds(k * L, L)
                sv2 = srcv[i, sl]
                gidx[i, sl] = sv2 + sv2 + coff
            return carry2

        lax.fori_loop(0, G, gid_body, 0)

        pltpu.async_copy(z_hbm.at[gidx.at[0]], zrows.at[0], gsem)
        pltpu.async_copy(z_hbm.at[gidx.at[1]], zrows.at[1], gsem)
        pltpu.async_copy(z_hbm.at[gidx.at[2]], zrows.at[2], gsem)

        def blk_body(b, carry2):
            cur = lax.rem(b, 5)
            pltpu.make_async_copy(
                z_hbm.at[gidx.at[b]], zrows.at[cur], gsem).wait()

            @pl.when(b >= 2)
            def _():
                drain = acc.at[dstv.at[b]]

                @pl.when(lax.rem(b, 2) == 0)
                def _():
                    pltpu.make_async_copy(
                        zrows.at[cur], drain, ssem0).wait()

                @pl.when(lax.rem(b, 2) == 1)
                def _():
                    pltpu.make_async_copy(
                        zrows.at[cur], drain, ssem1).wait()

            @pl.when(b + 3 < G)
            def _():
                pltpu.async_copy(
                    z_hbm.at[gidx.at[b + 3]], zrows.at[lax.rem(b + 3, 5)],
                    gsem)

            for k in range(B // L):
                sl = pl.ds(k * L, L)
                si = srcv[b, sl]
                di = dstv[b, sl]
                sv = plsc.load_gather(stab, [si])
                tv = plsc.load_gather(ttab, [di])
                e16 = jnp.exp(sv + tv)
                ev[b, sl] = e16
                for j in range(L):
                    av = e16.at[jnp.full((L,), j, jnp.int32)].get(
                        mode="promise_in_bounds")
                    r = k * L + j
                    for m in range(DH // L):
                        sl2 = pl.ds(m * L, L)
                        zrows[cur, r, sl2] = zrows[cur, r, sl2] * av

            @pl.when(lax.rem(b, 2) == 0)
            def _():
                pltpu.async_copy(
                    zrows.at[cur], acc.at[dstv.at[b]], ssem0, add=True)

            @pl.when(lax.rem(b, 2) == 1)
            def _():
                pltpu.async_copy(
                    zrows.at[cur], acc.at[dstv.at[b]], ssem1, add=True)

            @pl.when(c == 0)
            def _():
                pltpu.async_copy(ev.at[b], accd.at[dstv.at[b]], dsem,
                                 add=True)

            return carry2

        lax.fori_loop(0, G, blk_body, 0)

        @pl.when(c == 0)
        def _():
            pltpu.make_async_copy(src_hbm.at[s].at[pl.ds(0, G)],
                                  gidx, dsem).wait()

        pltpu.make_async_copy(
            zrows.at[0], acc.at[dstv.at[0]], ssem0).wait()
        pltpu.make_async_copy(
            zrows.at[0], acc.at[dstv.at[0]], ssem1).wait()
        return carry

    lax.fori_loop(0, NCHUNK, chunk_body, 0)
    plsc.subcore_barrier()

    rs = pl.ds(s * ROWS_PER_TILE, ROWS_PER_TILE)
    pltpu.sync_copy(acc.at[rs], acc_out.at[c].at[rs])

    @pl.when(c == 0)
    def _():
        pltpu.sync_copy(accd.at[rs], den_out.at[rs])


def _sc_agg(src3, dst3, st, zflat):
    mesh = plsc.VectorSubcoreMesh(
        core_axis_name="c", subcore_axis_name="s", num_cores=NC,
        num_subcores=NS)
    f = pl.kernel(
        _sc_body,
        out_type=(
            jax.ShapeDtypeStruct((NC, N_ACC, DH), jnp.float32),
            jax.ShapeDtypeStruct((N_ACC,), jnp.float32),
        ),
        mesh=mesh,
        compiler_params=pltpu.CompilerParams(
            needs_layout_passes=False, use_tc_tiling_on_sc=False),
        scratch_types=[
            pltpu.VMEM((N_PAD,), jnp.float32),
            pltpu.VMEM((N_PAD,), jnp.float32),
            pltpu.VMEM((G, B), jnp.int32),
            pltpu.VMEM((G, B), jnp.int32),
            pltpu.VMEM((G, B), jnp.int32),
            pltpu.VMEM((G, B), jnp.float32),
            pltpu.VMEM((5, B, DH), jnp.float32),
            pltpu.VMEM((((ROWS_PER_TILE + L - 1) // L) * L,), jnp.float32),
            pltpu.VMEM_SHARED((N_ACC, DH), jnp.float32),
            pltpu.VMEM_SHARED((N_ACC,), jnp.float32),
            pltpu.SemaphoreType.DMA,
            pltpu.SemaphoreType.DMA,
            pltpu.SemaphoreType.DMA,
            pltpu.SemaphoreType.DMA,
        ],
    )
    return f(src3, dst3, st, zflat)


def _fin_body(acc_ref, den_ref, out_ref):
    a = jnp.concatenate(
        [acc_ref[0, :N_NODES], acc_ref[1, :N_NODES]], axis=1)
    d = den_ref[:N_NODES]
    h = a / jnp.maximum(d, 1e-16)[:, None]
    un = jnp.maximum(
        jnp.sqrt(jnp.sum(h * h, axis=1, keepdims=True)), MIN_NORM)
    o1 = jnp.tanh(un) * h / un
    n1 = jnp.maximum(
        jnp.sqrt(jnp.sum(o1 * o1, axis=1, keepdims=True)), MIN_NORM)
    maxnorm = 1.0 - BALL_EPS
    out_ref[...] = jnp.where(n1 > maxnorm, o1 / n1 * maxnorm, o1)


def _tc_finish(acc, den):
    return pl.pallas_call(
        _fin_body,
        out_shape=jax.ShapeDtypeStruct((N_NODES, D), jnp.float32),
    )(acc, den)


@jax.jit
def kernel(x, edge_index, attn_w):
    z, st = _tc_prep(x, attn_w)
    zflat = z.reshape(NC * N_PAD, DH)

    pad = N_NODES + (jnp.arange(EPS - EPS_RAW, dtype=jnp.int32)
                     % (N_ACC - N_NODES))
    pad2 = jnp.broadcast_to(pad, (NS, EPS - EPS_RAW))
    src3 = jnp.concatenate(
        [edge_index[0].reshape(NS, EPS_RAW), pad2], axis=1
    ).reshape(NS, NBLK, B)
    dst3 = jnp.concatenate(
        [edge_index[1].reshape(NS, EPS_RAW), pad2], axis=1
    ).reshape(NS, NBLK, B)

    acc, den = _sc_agg(src3, dst3, st, zflat)
    return _tc_finish(acc, den)
